# baseline (speedup 1.0000x reference)
import dataclasses
import functools

import jax
import jax.numpy as jnp
from jax import lax
from jax.experimental import pallas as pl
from jax.experimental.pallas import tpu as pltpu
from jax.experimental.pallas import tpu_sc as plsc

_NA = 12
_B = 16384
_BH = _B // 2
_A = 50
_A0 = 24
_D = 64
_NC = 2
_NS = 16
_NW = _NC * _NS
_COLS = _BH // _NW
_L = 16


def _sc_hist_body(xt_hbm, counts_hbm, x_v, acc_v, sem, sem2):
    wid = lax.axis_index("s") * _NC + lax.axis_index("c")
    base = wid * _COLS
    cp1 = pltpu.make_async_copy(
        xt_hbm.at[pl.ds(0, _A0), pl.ds(base, _COLS)],
        x_v.at[pl.ds(0, _A0)], sem)
    cp2 = pltpu.make_async_copy(
        xt_hbm.at[pl.ds(_A0, _A - _A0), pl.ds(base, _COLS)],
        x_v.at[pl.ds(_A0, _A - _A0)], sem2)
    cp1.start()
    cp2.start()
    cp1.wait()

    zeros = jnp.zeros((_L,), jnp.float32)
    ones = jnp.ones((_L,), jnp.float32)
    lane = lax.iota(jnp.int32, _L)

    @plsc.parallel_loop(0, _COLS, _L, unroll=4)
    def _(j):
        sl = pl.ds(j, _L)
        for a in range(_NA):
            acc_v[a, sl] = zeros
        col = lane + j
        for i in range(_A0):
            plsc.addupdate_scatter(acc_v, [x_v[i, sl], col], ones)

    cp2.wait()

    @plsc.parallel_loop(0, _COLS, _L, unroll=4)
    def _(j):
        sl = pl.ds(j, _L)
        col = lane + j
        for i in range(_A0, _A):
            plsc.addupdate_scatter(acc_v, [x_v[i, sl], col], ones)

    pltpu.async_copy(acc_v, counts_hbm.at[:, pl.ds(base, _COLS)], sem).wait()


def _make_sc_hist():
    mesh = plsc.VectorSubcoreMesh(core_axis_name="c", subcore_axis_name="s")
    cp = pltpu.CompilerParams()
    if "needs_layout_passes" in pltpu.CompilerParams.__dataclass_fields__:
        cp = dataclasses.replace(cp, needs_layout_passes=False)
    return pl.kernel(
        _sc_hist_body,
        out_type=jax.ShapeDtypeStruct((_NA, _BH), jnp.float32),
        mesh=mesh,
        scratch_types=[
            pltpu.VMEM((_A, _COLS), jnp.int32),
            pltpu.VMEM((_NA, _COLS), jnp.float32),
            pltpu.SemaphoreType.DMA,
            pltpu.SemaphoreType.DMA,
        ],
        compiler_params=cp,
    )


_MB = 4096


def _mm_body(c_ref, tbl_ref, o_ref):
    o_ref[...] = lax.dot_general(
        c_ref[...], tbl_ref[...], (((0,), (0,)), ((), ())),
        preferred_element_type=jnp.float32)


def _mm_half(counts, table):
    return pl.pallas_call(
        _mm_body,
        grid=(_BH // _MB,),
        in_specs=[
            pl.BlockSpec((_NA, _MB), lambda i: (0, i)),
            pl.BlockSpec((_NA, _D), lambda i: (0, 0)),
        ],
        out_specs=pl.BlockSpec((_MB, _D), lambda i: (i, 0)),
        out_shape=jax.ShapeDtypeStruct((_BH, _D), jnp.float32),
    )(counts, table)


def _mm2_body(o1_ref, c_ref, tbl_ref, o_ref):
    i = pl.program_id(0)
    nh = _BH // _MB

    @pl.when(i < nh)
    def _():
        o_ref[...] = o1_ref[...]

    @pl.when(i >= nh)
    def _():
        o_ref[...] = lax.dot_general(
            c_ref[...], tbl_ref[...], (((0,), (0,)), ((), ())),
            preferred_element_type=jnp.float32)


def _mm_final(out1, counts2, table):
    nh = _BH // _MB
    return pl.pallas_call(
        _mm2_body,
        grid=(2 * nh,),
        in_specs=[
            pl.BlockSpec((_MB, _D), lambda i: (i % nh, 0)),
            pl.BlockSpec((_NA, _MB), lambda i: (0, i % nh)),
            pl.BlockSpec((_NA, _D), lambda i: (0, 0)),
        ],
        out_specs=pl.BlockSpec((_MB, _D), lambda i: (i, 0)),
        out_shape=jax.ShapeDtypeStruct((_B, _D), jnp.float32),
    )(out1, counts2, table)


def kernel(input, action_table):
    x = input.astype(jnp.int32)
    xt1 = x[:_BH].T
    xt2 = x[_BH:].T
    sc = _make_sc_hist()
    counts1 = sc(xt1)
    out1 = _mm_half(counts1, action_table)
    counts2 = sc(xt2)
    return _mm_final(out1, counts2, action_table)


# speedup vs baseline: 1.1811x; 1.1811x over previous
import dataclasses
import functools

import jax
import jax.numpy as jnp
from jax import lax
from jax.experimental import pallas as pl
from jax.experimental.pallas import tpu as pltpu
from jax.experimental.pallas import tpu_sc as plsc

_NA = 12
_B = 16384
_A = 50
_D = 64
_NC = 2
_NS = 16
_NW = _NC * _NS
_COLS = _B // _NW
_L = 16


_A0 = 24


def _sc_hist_body(xt_hbm, counts_hbm, x_v, acc_v, sem, sem2):
    wid = lax.axis_index("s") * _NC + lax.axis_index("c")
    base = wid * _COLS
    cp1 = pltpu.make_async_copy(
        xt_hbm.at[pl.ds(0, _A0), pl.ds(base, _COLS)],
        x_v.at[pl.ds(0, _A0)], sem)
    cp2 = pltpu.make_async_copy(
        xt_hbm.at[pl.ds(_A0, _A - _A0), pl.ds(base, _COLS)],
        x_v.at[pl.ds(_A0, _A - _A0)], sem2)
    cp1.start()
    cp2.start()

    zeros = jnp.zeros((_L,), jnp.float32)
    ones = jnp.ones((_L,), jnp.float32)
    lane = lax.iota(jnp.int32, _L)

    @plsc.parallel_loop(0, _COLS, _L, unroll=4)
    def _(j):
        sl = pl.ds(j, _L)
        for a in range(_NA):
            acc_v[a, sl] = zeros

    cp1.wait()

    @plsc.parallel_loop(0, _COLS, _L, unroll=4)
    def _(j):
        sl = pl.ds(j, _L)
        col = lane + j
        for i in range(_A0):
            plsc.addupdate_scatter(acc_v, [x_v[i, sl], col], ones)

    cp2.wait()

    @plsc.parallel_loop(0, _COLS, _L, unroll=4)
    def _(j):
        sl = pl.ds(j, _L)
        col = lane + j
        for i in range(_A0, _A):
            plsc.addupdate_scatter(acc_v, [x_v[i, sl], col], ones)

    pltpu.async_copy(acc_v, counts_hbm.at[:, pl.ds(base, _COLS)], sem).wait()


@jax.jit
def _sc_hist(xt):
    mesh = plsc.VectorSubcoreMesh(core_axis_name="c", subcore_axis_name="s")
    cp = pltpu.CompilerParams()
    if "needs_layout_passes" in pltpu.CompilerParams.__dataclass_fields__:
        cp = dataclasses.replace(cp, needs_layout_passes=False)
    f = pl.kernel(
        _sc_hist_body,
        out_type=jax.ShapeDtypeStruct((_NA, _B), jnp.float32),
        mesh=mesh,
        scratch_types=[
            pltpu.VMEM((_A, _COLS), jnp.int32),
            pltpu.VMEM((_NA, _COLS), jnp.float32),
            pltpu.SemaphoreType.DMA,
            pltpu.SemaphoreType.DMA,
        ],
        compiler_params=cp,
    )
    return f(xt)


def _mm_body(c_ref, tbl_ref, o_ref):
    o_ref[...] = lax.dot_general(
        c_ref[...], tbl_ref[...], (((0,), (0,)), ((), ())),
        preferred_element_type=jnp.float32)


_MB = 4096


def kernel(input, action_table):
    xt = input.astype(jnp.int32).T
    counts = _sc_hist(xt)
    return pl.pallas_call(
        _mm_body,
        grid=(_B // _MB,),
        in_specs=[
            pl.BlockSpec((_NA, _MB), lambda i: (0, i)),
            pl.BlockSpec((_NA, _D), lambda i: (0, 0)),
        ],
        out_specs=pl.BlockSpec((_MB, _D), lambda i: (i, 0)),
        out_shape=jax.ShapeDtypeStruct((_B, _D), jnp.float32),
    )(counts, action_table)


# speedup vs baseline: 1.2400x; 1.0499x over previous
import dataclasses

import jax
import jax.numpy as jnp
from jax import lax
from jax.experimental import pallas as pl
from jax.experimental.pallas import tpu as pltpu
from jax.experimental.pallas import tpu_sc as plsc

_NA = 12
_B = 16384
_A = 50
_D = 64
_NC = 2
_NS = 16
_NW = _NC * _NS
_COLS = _B // _NW
_L = 16


_A0 = 24


def _sc_hist_body(xt_hbm, counts_hbm, x_v, acc_v, sem, sem2):
    wid = lax.axis_index("s") * _NC + lax.axis_index("c")
    base = wid * _COLS
    cp1 = pltpu.make_async_copy(
        xt_hbm.at[pl.ds(0, _A0), pl.ds(base, _COLS)],
        x_v.at[pl.ds(0, _A0)], sem)
    cp2 = pltpu.make_async_copy(
        xt_hbm.at[pl.ds(_A0, _A - _A0), pl.ds(base, _COLS)],
        x_v.at[pl.ds(_A0, _A - _A0)], sem2)
    cp1.start()
    cp2.start()
    cp1.wait()

    zeros = jnp.zeros((_L,), jnp.float32)
    ones = jnp.ones((_L,), jnp.float32)
    lane = lax.iota(jnp.int32, _L)

    @plsc.parallel_loop(0, _COLS, _L, unroll=4)
    def _(j):
        sl = pl.ds(j, _L)
        for a in range(_NA):
            acc_v[a, sl] = zeros
        col = lane + j
        for i in range(_A0):
            plsc.addupdate_scatter(acc_v, [x_v[i, sl], col], ones)

    cp2.wait()

    @plsc.parallel_loop(0, _COLS, _L, unroll=4)
    def _(j):
        sl = pl.ds(j, _L)
        col = lane + j
        for i in range(_A0, _A):
            plsc.addupdate_scatter(acc_v, [x_v[i, sl], col], ones)

    pltpu.async_copy(acc_v, counts_hbm.at[:, pl.ds(base, _COLS)], sem).wait()


@jax.jit
def _sc_hist(xt):
    mesh = plsc.VectorSubcoreMesh(core_axis_name="c", subcore_axis_name="s")
    cp = pltpu.CompilerParams()
    if "needs_layout_passes" in pltpu.CompilerParams.__dataclass_fields__:
        cp = dataclasses.replace(cp, needs_layout_passes=False)
    f = pl.kernel(
        _sc_hist_body,
        out_type=jax.ShapeDtypeStruct((_NA, _B), jnp.float32),
        mesh=mesh,
        scratch_types=[
            pltpu.VMEM((_A, _COLS), jnp.int32),
            pltpu.VMEM((_NA, _COLS), jnp.float32),
            pltpu.SemaphoreType.DMA,
            pltpu.SemaphoreType.DMA,
        ],
        compiler_params=cp,
    )
    return f(xt)


def _mm_body(c_ref, tbl_ref, o_ref):
    o_ref[...] = lax.dot_general(
        c_ref[...], tbl_ref[...], (((0,), (0,)), ((), ())),
        preferred_element_type=jnp.float32)


_MB = 4096


def kernel(input, action_table):
    xt = input.astype(jnp.int32).T
    counts = _sc_hist(xt)
    return pl.pallas_call(
        _mm_body,
        grid=(_B // _MB,),
        in_specs=[
            pl.BlockSpec((_NA, _MB), lambda i: (0, i)),
            pl.BlockSpec((_NA, _D), lambda i: (0, 0)),
        ],
        out_specs=pl.BlockSpec((_MB, _D), lambda i: (i, 0)),
        out_shape=jax.ShapeDtypeStruct((_B, _D), jnp.float32),
    )(counts, action_table)
